# baseline (speedup 1.0000x reference)
import jax
import jax.numpy as jnp
from jax import lax
from jax.experimental import pallas as pl
from jax.experimental.pallas import tpu as pltpu

SIGMA = 5.0
B, C, H, W = 128, 3, 384, 384
INV = 1.0 / (2.0 * SIGMA * SIGMA)


def _bulk_copy(b, mask_ref, img_ref, out_ref, sem, do):
    @pl.when(mask_ref[b] == 0)
    def _():
        cp = pltpu.make_async_copy(
            img_ref.at[pl.ds(b, 1)], out_ref.at[pl.ds(b, 1)], sem)
        cp.start() if do else cp.wait()

    @pl.when(mask_ref[b] != 0)
    def _():
        cp = pltpu.make_async_copy(
            img_ref.at[pl.ds(b, 1), pl.ds(1, 2)],
            out_ref.at[pl.ds(b, 1), pl.ds(1, 2)], sem)
        cp.start() if do else cp.wait()


def _body(mask_ref, gt_ref, img_ref, out_ref, heat_ref, bulk_sem, heat_sem):
    def fire(b, c):
        _bulk_copy(b, mask_ref, img_ref, out_ref, bulk_sem, do=True)
        return c
    lax.fori_loop(0, B, fire, 0)

    def heat(b, c):
        @pl.when(mask_ref[b] != 0)
        def _():
            x0 = gt_ref[b, 0]
            y0 = gt_ref[b, 1]
            xs = lax.broadcasted_iota(jnp.int32, (1, 1, 1, W), 3).astype(jnp.float32)
            ys = lax.broadcasted_iota(jnp.int32, (1, 1, H, 1), 2).astype(jnp.float32)
            gx = jnp.exp(-((xs - x0) ** 2) * INV)
            gy = jnp.exp(-((ys - y0) ** 2) * INV)
            heat_ref[...] = gy * gx
            cp = pltpu.make_async_copy(
                heat_ref, out_ref.at[pl.ds(b, 1), pl.ds(0, 1)], heat_sem)
            cp.start()
            cp.wait()
        return c
    lax.fori_loop(0, B, heat, 0)

    def drain(b, c):
        _bulk_copy(b, mask_ref, img_ref, out_ref, bulk_sem, do=False)
        return c
    lax.fori_loop(0, B, drain, 0)


def kernel(images, gt, mask):
    mask_i = mask.astype(jnp.int32)
    return pl.pallas_call(
        _body,
        grid=(1,),
        in_specs=[
            pl.BlockSpec(memory_space=pltpu.SMEM),
            pl.BlockSpec(memory_space=pltpu.SMEM),
            pl.BlockSpec(memory_space=pl.ANY),
        ],
        out_specs=pl.BlockSpec(memory_space=pl.ANY),
        out_shape=jax.ShapeDtypeStruct((B, C, H, W), jnp.float32),
        scratch_shapes=[
            pltpu.VMEM((1, 1, H, W), jnp.float32),
            pltpu.SemaphoreType.DMA,
            pltpu.SemaphoreType.DMA,
        ],
    )(mask_i, gt, images)


# speedup vs baseline: 39.9390x; 39.9390x over previous
import jax
import jax.numpy as jnp
from jax import lax
from jax.experimental import pallas as pl
from jax.experimental.pallas import tpu as pltpu

SIGMA = 5.0
B, C, H, W = 128, 3, 384, 384
INV = 1.0 / (2.0 * SIGMA * SIGMA)


def _body(mask_ref, gt_ref, img_ref, out_ref):
    b = pl.program_id(0)
    out_ref[...] = img_ref[...]

    @pl.when(mask_ref[b] != 0)
    def _():
        x0 = gt_ref[b, 0]
        y0 = gt_ref[b, 1]
        xs = lax.broadcasted_iota(jnp.int32, (1, W), 1).astype(jnp.float32)
        ys = lax.broadcasted_iota(jnp.int32, (H, 1), 0).astype(jnp.float32)
        gx = jnp.exp(-((xs - x0) ** 2) * INV)
        gy = jnp.exp(-((ys - y0) ** 2) * INV)
        out_ref[0, 0] = gy * gx


def kernel(images, gt, mask):
    mask_i = mask.astype(jnp.int32)
    return pl.pallas_call(
        _body,
        grid=(B,),
        in_specs=[
            pl.BlockSpec(memory_space=pltpu.SMEM),
            pl.BlockSpec(memory_space=pltpu.SMEM),
            pl.BlockSpec((1, C, H, W), lambda b: (b, 0, 0, 0)),
        ],
        out_specs=pl.BlockSpec((1, C, H, W), lambda b: (b, 0, 0, 0)),
        out_shape=jax.ShapeDtypeStruct((B, C, H, W), jnp.float32),
    )(mask_i, gt, images)


# speedup vs baseline: 46.5323x; 1.1651x over previous
import jax
import jax.numpy as jnp
from jax import lax
from jax.experimental import pallas as pl
from jax.experimental.pallas import tpu as pltpu

SIGMA = 5.0
B, C, H, W = 128, 3, 384, 384
INV = 1.0 / (2.0 * SIGMA * SIGMA)
BB = 4


def _body(mask_ref, gt_ref, img_ref, out_ref):
    i = pl.program_id(0)
    out_ref[...] = img_ref[...]

    for j in range(BB):
        b = i * BB + j

        @pl.when(mask_ref[b] != 0)
        def _():
            x0 = gt_ref[b, 0]
            y0 = gt_ref[b, 1]
            xs = lax.broadcasted_iota(jnp.int32, (1, W), 1).astype(jnp.float32)
            ys = lax.broadcasted_iota(jnp.int32, (H, 1), 0).astype(jnp.float32)
            gx = jnp.exp(-((xs - x0) ** 2) * INV)
            gy = jnp.exp(-((ys - y0) ** 2) * INV)
            out_ref[j, 0] = gy * gx


def kernel(images, gt, mask):
    mask_i = mask.astype(jnp.int32)
    return pl.pallas_call(
        _body,
        grid=(B // BB,),
        in_specs=[
            pl.BlockSpec(memory_space=pltpu.SMEM),
            pl.BlockSpec(memory_space=pltpu.SMEM),
            pl.BlockSpec((BB, C, H, W), lambda i: (i, 0, 0, 0)),
        ],
        out_specs=pl.BlockSpec((BB, C, H, W), lambda i: (i, 0, 0, 0)),
        out_shape=jax.ShapeDtypeStruct((B, C, H, W), jnp.float32),
    )(mask_i, gt, images)


# speedup vs baseline: 46.6906x; 1.0034x over previous
import jax
import jax.numpy as jnp
from jax import lax
from jax.experimental import pallas as pl
from jax.experimental.pallas import tpu as pltpu

SIGMA = 5.0
B, C, H, W = 128, 3, 384, 384
INV = 1.0 / (2.0 * SIGMA * SIGMA)
BB = 8


def _body(mask_ref, gt_ref, img_ref, out_ref):
    i = pl.program_id(0)
    out_ref[...] = img_ref[...]

    for j in range(BB):
        b = i * BB + j

        @pl.when(mask_ref[b] != 0)
        def _():
            x0 = gt_ref[b, 0]
            y0 = gt_ref[b, 1]
            xs = lax.broadcasted_iota(jnp.int32, (1, W), 1).astype(jnp.float32)
            ys = lax.broadcasted_iota(jnp.int32, (H, 1), 0).astype(jnp.float32)
            gx = jnp.exp(-((xs - x0) ** 2) * INV)
            gy = jnp.exp(-((ys - y0) ** 2) * INV)
            out_ref[j, 0] = gy * gx


def kernel(images, gt, mask):
    mask_i = mask.astype(jnp.int32)
    return pl.pallas_call(
        _body,
        grid=(B // BB,),
        in_specs=[
            pl.BlockSpec(memory_space=pltpu.SMEM),
            pl.BlockSpec(memory_space=pltpu.SMEM),
            pl.BlockSpec((BB, C, H, W), lambda i: (i, 0, 0, 0)),
        ],
        out_specs=pl.BlockSpec((BB, C, H, W), lambda i: (i, 0, 0, 0)),
        out_shape=jax.ShapeDtypeStruct((B, C, H, W), jnp.float32),
    )(mask_i, gt, images)
